# baseline (speedup 1.0000x reference)
import jax
import jax.numpy as jnp
from jax import lax
from jax.experimental import pallas as pl
from jax.experimental.pallas import tpu as pltpu
from jax.experimental.pallas import tpu_sc as plsc

N = 10000
E = 320000
D = 128
DH = D // 2
NC = 2
NS = 16
CHUNK = 128
NCHUNK = 160
E_PAD = NS * NCHUNK * CHUNK
EPT = E_PAD // NS
NDUM = 128
NSLOT = 4
OFF = 2
NHALF = 4
HCHUNK = NCHUNK // NHALF
NWAVE = HCHUNK // NSLOT
BLK = 2000


def _sc_agg(h, cols, rows):
    mesh = plsc.VectorSubcoreMesh(core_axis_name="c", subcore_axis_name="s")

    def body(h_hbm, cols_hbm, rows_hbm, out_hbm,
             cols_v, rows_v, bufs, h_sh, agg_sh, *sems):
        gsems = sems[:NSLOT]
        ssems = sems[NSLOT:]
        c = lax.axis_index("c")
        s = lax.axis_index("s")
        off = s * 624

        pltpu.sync_copy(h_hbm.at[pl.ds(off, 640), pl.ds(c * DH, DH)],
                        h_sh.at[pl.ds(off, 640)])

        def zrow(i, carry):
            for k in range(DH // 16):
                bufs[0, i, pl.ds(k * 16, 16)] = jnp.zeros((16,), jnp.float32)
            return carry
        lax.fori_loop(0, CHUNK, zrow, 0)
        for k in range(5):
            pltpu.sync_copy(bufs.at[0],
                            agg_sh.at[pl.ds(off + k * CHUNK, CHUNK)])
        plsc.subcore_barrier()

        def gstart(slot, j):
            pltpu.async_copy(h_sh.at[cols_v.at[pl.ds(j * CHUNK, CHUNK)]],
                             bufs.at[slot], gsems[slot])

        def gwait(slot):
            pltpu.make_async_copy(h_sh.at[cols_v.at[pl.ds(0, CHUNK)]],
                                  bufs.at[slot], gsems[slot]).wait()

        def sstart(slot, j):
            pltpu.async_copy(bufs.at[slot],
                             agg_sh.at[rows_v.at[pl.ds(j * CHUNK, CHUNK)]],
                             ssems[slot], add=True)

        def swait(slot):
            pltpu.make_async_copy(bufs.at[slot],
                                  agg_sh.at[rows_v.at[pl.ds(0, CHUNK)]],
                                  ssems[slot]).wait()

        for half in range(NHALF):
            gbase = s * EPT + half * (HCHUNK * CHUNK)
            pltpu.sync_copy(cols_hbm.at[pl.ds(gbase, HCHUNK * CHUNK)], cols_v)
            pltpu.sync_copy(rows_hbm.at[pl.ds(gbase, HCHUNK * CHUNK)], rows_v)

            for k in range(NSLOT):
                gstart(k, k)

            def wave(w, carry):
                base = w * NSLOT

                def regather(kk):
                    swait(kk)

                    @pl.when(w + 1 < NWAVE)
                    def _():
                        gstart(kk, base + NSLOT + kk)

                for k in range(NSLOT):
                    gwait(k)
                    sstart(k, base + k)
                    if k >= OFF:
                        regather(k - OFF)
                for kk in range(NSLOT - OFF, NSLOT):
                    regather(kk)
                return carry
            lax.fori_loop(0, NWAVE, wave, 0)

        plsc.subcore_barrier()
        pltpu.sync_copy(agg_sh.at[pl.ds(off, 640)],
                        out_hbm.at[c].at[pl.ds(off, 640)])

    f = pl.kernel(
        body,
        out_type=jax.ShapeDtypeStruct((NC, N, DH), jnp.float32),
        mesh=mesh,
        scratch_types=[
            pltpu.VMEM((HCHUNK * CHUNK,), jnp.int32),
            pltpu.VMEM((HCHUNK * CHUNK,), jnp.int32),
            pltpu.VMEM((NSLOT, CHUNK, DH), jnp.float32),
            pltpu.VMEM_SHARED((N, DH), jnp.float32),
            pltpu.VMEM_SHARED((N + NDUM, DH), jnp.float32),
        ] + [pltpu.SemaphoreType.DMA] * (2 * NSLOT),
        compiler_params=pltpu.CompilerParams(use_tc_tiling_on_sc=False),
    )
    return f(h, cols, rows)


def _tc_layer(p, h, W, S, b):
    def body(p_ref, h_ref, w_ref, s_ref, b_ref, o_ref):
        agg = jnp.concatenate([p_ref[0], p_ref[1]], axis=1)
        acc = jnp.dot(agg, w_ref[...], preferred_element_type=jnp.float32)
        acc += jnp.dot(h_ref[...], s_ref[...], preferred_element_type=jnp.float32)
        acc += b_ref[...]
        o_ref[...] = jnp.maximum(acc, 0.0)

    nb = N // BLK
    return pl.pallas_call(
        body,
        grid=(nb,),
        in_specs=[
            pl.BlockSpec((NC, BLK, DH), lambda i: (0, i, 0)),
            pl.BlockSpec((BLK, D), lambda i: (i, 0)),
            pl.BlockSpec((D, D), lambda i: (0, 0)),
            pl.BlockSpec((D, D), lambda i: (0, 0)),
            pl.BlockSpec((1, D), lambda i: (0, 0)),
        ],
        out_specs=pl.BlockSpec((BLK, D), lambda i: (i, 0)),
        out_shape=jax.ShapeDtypeStruct((N, D), jnp.float32),
    )(p, h, W, S, b.reshape(1, D))


def _tc_layer_final(p, h, W, S, b, Wf, bf):
    def body(p_ref, h_ref, w_ref, s_ref, b_ref, wf_ref, bf_ref, o_ref):
        agg = jnp.concatenate([p_ref[0], p_ref[1]], axis=1)
        acc = jnp.dot(agg, w_ref[...], preferred_element_type=jnp.float32)
        acc += jnp.dot(h_ref[...], s_ref[...], preferred_element_type=jnp.float32)
        acc += b_ref[...]
        t = jnp.maximum(acc, 0.0)
        out = lax.dot_general(t, wf_ref[...], (((1,), (1,)), ((), ())),
                              preferred_element_type=jnp.float32)
        o_ref[...] = out + bf_ref[...]

    nb = N // BLK
    return pl.pallas_call(
        body,
        grid=(nb,),
        in_specs=[
            pl.BlockSpec((NC, BLK, DH), lambda i: (0, i, 0)),
            pl.BlockSpec((BLK, D), lambda i: (i, 0)),
            pl.BlockSpec((D, D), lambda i: (0, 0)),
            pl.BlockSpec((D, D), lambda i: (0, 0)),
            pl.BlockSpec((1, D), lambda i: (0, 0)),
            pl.BlockSpec((D, D), lambda i: (0, 0)),
            pl.BlockSpec((1, D), lambda i: (0, 0)),
        ],
        out_specs=pl.BlockSpec((BLK, D), lambda i: (i, 0)),
        out_shape=jax.ShapeDtypeStruct((N, D), jnp.float32),
    )(p, h, W, S, b.reshape(1, D), Wf, bf.reshape(1, D))


def _tc_split_edges(edge_index):
    npadrow = (E_PAD - E) // 128

    def body(e_ref, r_ref, c_ref):
        rpad = N + lax.broadcasted_iota(jnp.int32, (npadrow, 128), 1) % NDUM
        r_ref[...] = jnp.concatenate(
            [e_ref[0].reshape(E // 128, 128), rpad], axis=0)
        c_ref[...] = jnp.concatenate(
            [e_ref[1].reshape(E // 128, 128),
             jnp.zeros((npadrow, 128), jnp.int32)], axis=0)

    return pl.pallas_call(
        body,
        out_shape=(jax.ShapeDtypeStruct((E_PAD // 128, 128), jnp.int32),
                   jax.ShapeDtypeStruct((E_PAD // 128, 128), jnp.int32)),
    )(edge_index)


def kernel(x, edge_index, W1, S1, b1, W2, S2, b2, Wf, bf):
    rows_2d, cols_2d = _tc_split_edges(edge_index)
    rows_p = rows_2d.reshape(E_PAD)
    cols_p = cols_2d.reshape(E_PAD)

    p1 = _sc_agg(x, cols_p, rows_p)
    h1 = _tc_layer(p1, x, W1, S1, b1)
    p2 = _sc_agg(h1, cols_p, rows_p)
    return _tc_layer_final(p2, h1, W2, S2, b2, Wf, bf)


# speedup vs baseline: 1.0855x; 1.0855x over previous
import jax
import jax.numpy as jnp
from jax import lax
from jax.experimental import pallas as pl
from jax.experimental.pallas import tpu as pltpu
from jax.experimental.pallas import tpu_sc as plsc

N = 10000
E = 320000
D = 128
DH = D // 2
NC = 2
NS = 16
CHUNK = 80
NCHUNK = 250
EPT = E // NS
NSLOT = 5
OFF = 3
NHALF = 2
HCHUNK = NCHUNK // NHALF
NWAVE = HCHUNK // NSLOT
BLK = 2000


def _sc_agg(h, cols, rows):
    mesh = plsc.VectorSubcoreMesh(core_axis_name="c", subcore_axis_name="s")

    def body(h_hbm, cols_hbm, rows_hbm, out_hbm,
             cols_v, rows_v, bufs, h_sh, agg_sh, *sems):
        gsems = sems[:NSLOT]
        ssems = sems[NSLOT:]
        c = lax.axis_index("c")
        s = lax.axis_index("s")
        off = s * 624

        pltpu.sync_copy(h_hbm.at[pl.ds(off, 640), pl.ds(c * DH, DH)],
                        h_sh.at[pl.ds(off, 640)])

        def zrow(i, carry):
            for k in range(DH // 16):
                bufs[0, i, pl.ds(k * 16, 16)] = jnp.zeros((16,), jnp.float32)
            return carry
        lax.fori_loop(0, CHUNK, zrow, 0)
        for k in range(8):
            pltpu.sync_copy(bufs.at[0],
                            agg_sh.at[pl.ds(off + k * CHUNK, CHUNK)])
        plsc.subcore_barrier()

        def gstart(slot, j):
            pltpu.async_copy(h_sh.at[cols_v.at[pl.ds(j * CHUNK, CHUNK)]],
                             bufs.at[slot], gsems[slot])

        def gwait(slot):
            pltpu.make_async_copy(h_sh.at[cols_v.at[pl.ds(0, CHUNK)]],
                                  bufs.at[slot], gsems[slot]).wait()

        def sstart(slot, j):
            pltpu.async_copy(bufs.at[slot],
                             agg_sh.at[rows_v.at[pl.ds(j * CHUNK, CHUNK)]],
                             ssems[slot], add=True)

        def swait(slot):
            pltpu.make_async_copy(bufs.at[slot],
                                  agg_sh.at[rows_v.at[pl.ds(0, CHUNK)]],
                                  ssems[slot]).wait()

        for half in range(NHALF):
            gbase = s * EPT + half * (HCHUNK * CHUNK)
            pltpu.sync_copy(cols_hbm.at[pl.ds(gbase, HCHUNK * CHUNK)], cols_v)
            pltpu.sync_copy(rows_hbm.at[pl.ds(gbase, HCHUNK * CHUNK)], rows_v)

            for k in range(NSLOT):
                gstart(k, k)

            def wave(w, carry):
                base = w * NSLOT

                def regather(kk):
                    swait(kk)

                    @pl.when(w + 1 < NWAVE)
                    def _():
                        gstart(kk, base + NSLOT + kk)

                for k in range(NSLOT):
                    gwait(k)
                    sstart(k, base + k)
                    if k >= OFF:
                        regather(k - OFF)
                for kk in range(NSLOT - OFF, NSLOT):
                    regather(kk)
                return carry
            lax.fori_loop(0, NWAVE, wave, 0)

        plsc.subcore_barrier()
        pltpu.sync_copy(agg_sh.at[pl.ds(off, 640)],
                        out_hbm.at[c].at[pl.ds(off, 640)])

    f = pl.kernel(
        body,
        out_type=jax.ShapeDtypeStruct((NC, N, DH), jnp.float32),
        mesh=mesh,
        scratch_types=[
            pltpu.VMEM((HCHUNK * CHUNK,), jnp.int32),
            pltpu.VMEM((HCHUNK * CHUNK,), jnp.int32),
            pltpu.VMEM((NSLOT, CHUNK, DH), jnp.float32),
            pltpu.VMEM_SHARED((N, DH), jnp.float32),
            pltpu.VMEM_SHARED((N, DH), jnp.float32),
        ] + [pltpu.SemaphoreType.DMA] * (2 * NSLOT),
        compiler_params=pltpu.CompilerParams(use_tc_tiling_on_sc=False),
    )
    return f(h, cols, rows)


def _tc_layer(p, h, W, S, b):
    def body(p_ref, h_ref, w_ref, s_ref, b_ref, o_ref):
        agg = jnp.concatenate([p_ref[0], p_ref[1]], axis=1)
        acc = jnp.dot(agg, w_ref[...], preferred_element_type=jnp.float32)
        acc += jnp.dot(h_ref[...], s_ref[...], preferred_element_type=jnp.float32)
        acc += b_ref[...]
        o_ref[...] = jnp.maximum(acc, 0.0)

    nb = N // BLK
    return pl.pallas_call(
        body,
        grid=(nb,),
        in_specs=[
            pl.BlockSpec((NC, BLK, DH), lambda i: (0, i, 0)),
            pl.BlockSpec((BLK, D), lambda i: (i, 0)),
            pl.BlockSpec((D, D), lambda i: (0, 0)),
            pl.BlockSpec((D, D), lambda i: (0, 0)),
            pl.BlockSpec((1, D), lambda i: (0, 0)),
        ],
        out_specs=pl.BlockSpec((BLK, D), lambda i: (i, 0)),
        out_shape=jax.ShapeDtypeStruct((N, D), jnp.float32),
    )(p, h, W, S, b.reshape(1, D))


def _tc_layer_final(p, h, W, S, b, Wf, bf):
    def body(p_ref, h_ref, w_ref, s_ref, b_ref, wf_ref, bf_ref, o_ref):
        agg = jnp.concatenate([p_ref[0], p_ref[1]], axis=1)
        acc = jnp.dot(agg, w_ref[...], preferred_element_type=jnp.float32)
        acc += jnp.dot(h_ref[...], s_ref[...], preferred_element_type=jnp.float32)
        acc += b_ref[...]
        t = jnp.maximum(acc, 0.0)
        out = lax.dot_general(t, wf_ref[...], (((1,), (1,)), ((), ())),
                              preferred_element_type=jnp.float32)
        o_ref[...] = out + bf_ref[...]

    nb = N // BLK
    return pl.pallas_call(
        body,
        grid=(nb,),
        in_specs=[
            pl.BlockSpec((NC, BLK, DH), lambda i: (0, i, 0)),
            pl.BlockSpec((BLK, D), lambda i: (i, 0)),
            pl.BlockSpec((D, D), lambda i: (0, 0)),
            pl.BlockSpec((D, D), lambda i: (0, 0)),
            pl.BlockSpec((1, D), lambda i: (0, 0)),
            pl.BlockSpec((D, D), lambda i: (0, 0)),
            pl.BlockSpec((1, D), lambda i: (0, 0)),
        ],
        out_specs=pl.BlockSpec((BLK, D), lambda i: (i, 0)),
        out_shape=jax.ShapeDtypeStruct((N, D), jnp.float32),
    )(p, h, W, S, b.reshape(1, D), Wf, bf.reshape(1, D))


def _tc_split_edges(edge_index):
    def body(e_ref, r_ref, c_ref):
        r_ref[...] = e_ref[0].reshape(r_ref.shape)
        c_ref[...] = e_ref[1].reshape(c_ref.shape)

    return pl.pallas_call(
        body,
        out_shape=(jax.ShapeDtypeStruct((E // 128, 128), jnp.int32),
                   jax.ShapeDtypeStruct((E // 128, 128), jnp.int32)),
    )(edge_index)


def kernel(x, edge_index, W1, S1, b1, W2, S2, b2, Wf, bf):
    rows_2d, cols_2d = _tc_split_edges(edge_index)
    rows_p = rows_2d.reshape(E)
    cols_p = cols_2d.reshape(E)

    p1 = _sc_agg(x, cols_p, rows_p)
    h1 = _tc_layer(p1, x, W1, S1, b1)
    p2 = _sc_agg(h1, cols_p, rows_p)
    return _tc_layer_final(p2, h1, W2, S2, b2, Wf, bf)


# speedup vs baseline: 1.2438x; 1.1458x over previous
import jax
import jax.numpy as jnp
from jax import lax
from jax.experimental import pallas as pl
from jax.experimental.pallas import tpu as pltpu
from jax.experimental.pallas import tpu_sc as plsc

N = 10000
E = 320000
D = 128
DH = D // 2
NC = 2
NS = 16
CHUNK = 80
NCHUNK = 250
EPT = E // NS
NSLOT = 5
OFF = 2
NHALF = 2
HCHUNK = NCHUNK // NHALF
NWAVE = HCHUNK // NSLOT
BLK = 2000


def _sc_agg(h, cols, rows):
    mesh = plsc.VectorSubcoreMesh(core_axis_name="c", subcore_axis_name="s")

    def body(h_hbm, cols_hbm, rows_hbm, out_hbm,
             cols_v, rows_v, bufs, h_sh, agg_sh, *sems):
        gsems = sems[:NSLOT]
        ssems = sems[NSLOT:]
        c = lax.axis_index("c")
        s = lax.axis_index("s")
        off = s * 624

        def zrow(i, carry):
            for k in range(DH // 16):
                bufs[0, i, pl.ds(k * 16, 16)] = jnp.zeros((16,), jnp.float32)
            return carry
        lax.fori_loop(0, CHUNK, zrow, 0)

        def prologue_copies(launch):
            yes = (pltpu.async_copy if launch
                   else lambda s_, d_, m: pltpu.make_async_copy(s_, d_, m).wait())
            yes(h_hbm.at[pl.ds(off, 640), pl.ds(c * DH, DH)],
                h_sh.at[pl.ds(off, 640)], gsems[0])
            for k in range(8):
                yes(bufs.at[0], agg_sh.at[pl.ds(off + k * CHUNK, CHUNK)],
                    ssems[0])
        prologue_copies(True)
        prologue_copies(False)
        plsc.subcore_barrier()

        def gstart(slot, j):
            pltpu.async_copy(h_sh.at[cols_v.at[pl.ds(j * CHUNK, CHUNK)]],
                             bufs.at[slot], gsems[slot])

        def gwait(slot):
            pltpu.make_async_copy(h_sh.at[cols_v.at[pl.ds(0, CHUNK)]],
                                  bufs.at[slot], gsems[slot]).wait()

        def sstart(slot, j):
            pltpu.async_copy(bufs.at[slot],
                             agg_sh.at[rows_v.at[pl.ds(j * CHUNK, CHUNK)]],
                             ssems[slot], add=True)

        def swait(slot):
            pltpu.make_async_copy(bufs.at[slot],
                                  agg_sh.at[rows_v.at[pl.ds(0, CHUNK)]],
                                  ssems[slot]).wait()

        for half in range(NHALF):
            gbase = s * EPT + half * (HCHUNK * CHUNK)
            pltpu.sync_copy(cols_hbm.at[pl.ds(gbase, HCHUNK * CHUNK)], cols_v)
            pltpu.sync_copy(rows_hbm.at[pl.ds(gbase, HCHUNK * CHUNK)], rows_v)

            for k in range(NSLOT):
                gstart(k, k)

            def wave(w, carry):
                base = w * NSLOT

                def regather(kk):
                    swait(kk)

                    @pl.when(w + 1 < NWAVE)
                    def _():
                        gstart(kk, base + NSLOT + kk)

                for k in range(NSLOT):
                    gwait(k)
                    sstart(k, base + k)
                    if k >= OFF:
                        regather(k - OFF)
                for kk in range(NSLOT - OFF, NSLOT):
                    regather(kk)
                return carry
            lax.fori_loop(0, NWAVE, wave, 0)

        plsc.subcore_barrier()
        pltpu.sync_copy(agg_sh.at[pl.ds(off, 640)],
                        out_hbm.at[c].at[pl.ds(off, 640)])

    f = pl.kernel(
        body,
        out_type=jax.ShapeDtypeStruct((NC, N, DH), jnp.float32),
        mesh=mesh,
        scratch_types=[
            pltpu.VMEM((HCHUNK * CHUNK,), jnp.int32),
            pltpu.VMEM((HCHUNK * CHUNK,), jnp.int32),
            pltpu.VMEM((NSLOT, CHUNK, DH), jnp.float32),
            pltpu.VMEM_SHARED((N, DH), jnp.float32),
            pltpu.VMEM_SHARED((N, DH), jnp.float32),
        ] + [pltpu.SemaphoreType.DMA] * (2 * NSLOT),
        compiler_params=pltpu.CompilerParams(use_tc_tiling_on_sc=False),
    )
    return f(h, cols, rows)


def _tc_layer(p, h, W, S, b):
    def body(p_ref, h_ref, w_ref, s_ref, b_ref, o_ref):
        agg = jnp.concatenate([p_ref[0], p_ref[1]], axis=1)
        acc = jnp.dot(agg, w_ref[...], preferred_element_type=jnp.float32)
        acc += jnp.dot(h_ref[...], s_ref[...], preferred_element_type=jnp.float32)
        acc += b_ref[...]
        o_ref[...] = jnp.maximum(acc, 0.0)

    nb = N // BLK
    return pl.pallas_call(
        body,
        grid=(nb,),
        in_specs=[
            pl.BlockSpec((NC, BLK, DH), lambda i: (0, i, 0)),
            pl.BlockSpec((BLK, D), lambda i: (i, 0)),
            pl.BlockSpec((D, D), lambda i: (0, 0)),
            pl.BlockSpec((D, D), lambda i: (0, 0)),
            pl.BlockSpec((1, D), lambda i: (0, 0)),
        ],
        out_specs=pl.BlockSpec((BLK, D), lambda i: (i, 0)),
        out_shape=jax.ShapeDtypeStruct((N, D), jnp.float32),
    )(p, h, W, S, b.reshape(1, D))


def _tc_layer_final(p, h, W, S, b, Wf, bf):
    def body(p_ref, h_ref, w_ref, s_ref, b_ref, wf_ref, bf_ref, o_ref):
        agg = jnp.concatenate([p_ref[0], p_ref[1]], axis=1)
        acc = jnp.dot(agg, w_ref[...], preferred_element_type=jnp.float32)
        acc += jnp.dot(h_ref[...], s_ref[...], preferred_element_type=jnp.float32)
        acc += b_ref[...]
        t = jnp.maximum(acc, 0.0)
        out = lax.dot_general(t, wf_ref[...], (((1,), (1,)), ((), ())),
                              preferred_element_type=jnp.float32)
        o_ref[...] = out + bf_ref[...]

    nb = N // BLK
    return pl.pallas_call(
        body,
        grid=(nb,),
        in_specs=[
            pl.BlockSpec((NC, BLK, DH), lambda i: (0, i, 0)),
            pl.BlockSpec((BLK, D), lambda i: (i, 0)),
            pl.BlockSpec((D, D), lambda i: (0, 0)),
            pl.BlockSpec((D, D), lambda i: (0, 0)),
            pl.BlockSpec((1, D), lambda i: (0, 0)),
            pl.BlockSpec((D, D), lambda i: (0, 0)),
            pl.BlockSpec((1, D), lambda i: (0, 0)),
        ],
        out_specs=pl.BlockSpec((BLK, D), lambda i: (i, 0)),
        out_shape=jax.ShapeDtypeStruct((N, D), jnp.float32),
    )(p, h, W, S, b.reshape(1, D), Wf, bf.reshape(1, D))


def _tc_split_edges(edge_index):
    def body(e_ref, r_ref, c_ref):
        r_ref[...] = e_ref[0].reshape(r_ref.shape)
        c_ref[...] = e_ref[1].reshape(c_ref.shape)

    return pl.pallas_call(
        body,
        out_shape=(jax.ShapeDtypeStruct((E // 128, 128), jnp.int32),
                   jax.ShapeDtypeStruct((E // 128, 128), jnp.int32)),
    )(edge_index)


def kernel(x, edge_index, W1, S1, b1, W2, S2, b2, Wf, bf):
    rows_2d, cols_2d = _tc_split_edges(edge_index)
    rows_p = rows_2d.reshape(E)
    cols_p = cols_2d.reshape(E)

    p1 = _sc_agg(x, cols_p, rows_p)
    h1 = _tc_layer(p1, x, W1, S1, b1)
    p2 = _sc_agg(h1, cols_p, rows_p)
    return _tc_layer_final(p2, h1, W2, S2, b2, Wf, bf)
